# baseline (speedup 1.0000x reference)
import jax
import jax.numpy as jnp
from jax import lax
from jax.experimental import pallas as pl
from jax.experimental.pallas import tpu as pltpu
from jax.experimental.pallas import tpu_sc as plsc

B0, B1 = 64, 32768
M = B0 * B1
JDIM, IDIM = 5, 4
E = JDIM * IDIM
K = 3
NC, NS, L = 2, 16, 16
NW = NC * NS
TPW = M // NW
C = 2048
NCHUNK = TPW // C
GROUPS = C // L


def _round_bf16(v):
    u = plsc.bitcast(v, jnp.int32)
    u = (u + 0x8000) & jnp.int32(-65536)
    return plsc.bitcast(u, jnp.float32)


def _sc_body(xf, wf, wbf, bf, vals, idxs, w_v, wb_v, b_v, in_v, vo_v, io_v):
    cid = lax.axis_index("c")
    sid = lax.axis_index("s")
    wid = sid * NC + cid
    pltpu.sync_copy(wf, w_v)
    pltpu.sync_copy(wbf, wb_v)
    pltpu.sync_copy(bf, b_v)
    lanes = lax.iota(jnp.int32, L)
    wsf = [w_v[pl.ds(k * L, L)] for k in range(E)]
    wsb = [wb_v[pl.ds(k * L, L)] for k in range(E)]
    bs = [b_v[pl.ds(o * L, L)] for o in range(JDIM)]

    def chunk_body(c, carry):
        base = wid * TPW + c * C
        pltpu.sync_copy(xf.at[pl.ds(base * E, C * E)], in_v)

        def group_body(g, carry):
            offs = (lanes + g * L) * E
            o3 = (lanes + g * L) * K
            s = []
            for j in range(JDIM):
                h = [_round_bf16(
                        plsc.load_gather(in_v, [offs + (j * IDIM + i)])
                        + wsf[j * IDIM + i])
                     for i in range(IDIM)]
                acc_sum = None
                for o in range(JDIM):
                    acc = bs[o]
                    for i in range(IDIM):
                        acc = acc + h[i] * wsb[o * IDIM + i]
                    acc = jnp.maximum(acc, 0.0)
                    acc_sum = acc if acc_sum is None else acc_sum + acc
                s.append(acc_sum)
            for k in range(K):
                bv = s[0]
                bi = jnp.zeros((L,), jnp.int32)
                for j in range(1, JDIM):
                    gt = s[j] > bv
                    bv = jnp.where(gt, s[j], bv)
                    bi = jnp.where(gt, j, bi)
                plsc.store_scatter(vo_v, [o3 + k], bv)
                plsc.store_scatter(io_v, [o3 + k], bi)
                if k < K - 1:
                    s = [jnp.where(bi == j, -1.0, s[j]) for j in range(JDIM)]
            return carry

        lax.fori_loop(0, GROUPS, group_body, 0)
        pltpu.sync_copy(vo_v, vals.at[pl.ds(base * K, C * K)])
        pltpu.sync_copy(io_v, idxs.at[pl.ds(base * K, C * K)])
        return carry

    lax.fori_loop(0, NCHUNK, chunk_body, 0)


def kernel(x, W, b):
    xf = x.reshape(M * E)
    wi = lax.bitcast_convert_type(W, jnp.int32)
    wi = (wi + 0x7FFF + ((wi >> 16) & 1)) & jnp.int32(-65536)
    Wb = lax.bitcast_convert_type(wi, jnp.float32)
    wf = jnp.repeat(W.reshape(E), L)
    wbf = jnp.repeat(Wb.reshape(E), L)
    bf = jnp.repeat(b, L)
    mesh = plsc.VectorSubcoreMesh(core_axis_name="c", subcore_axis_name="s")
    vals, idxs = pl.kernel(
        _sc_body,
        out_type=(jax.ShapeDtypeStruct((M * K,), jnp.float32),
                  jax.ShapeDtypeStruct((M * K,), jnp.int32)),
        mesh=mesh,
        compiler_params=pltpu.CompilerParams(needs_layout_passes=False),
        scratch_types=[
            pltpu.VMEM((E * L,), jnp.float32),
            pltpu.VMEM((E * L,), jnp.float32),
            pltpu.VMEM((JDIM * L,), jnp.float32),
            pltpu.VMEM((C * E,), jnp.float32),
            pltpu.VMEM((C * K,), jnp.float32),
            pltpu.VMEM((C * K,), jnp.int32),
        ],
    )(xf, wf, wbf, bf)
    return vals.reshape(B0, B1, K), idxs.reshape(B0, B1, K)


# speedup vs baseline: 5.8361x; 5.8361x over previous
import jax
import jax.numpy as jnp
from jax import lax
from jax.experimental import pallas as pl
from jax.experimental.pallas import tpu as pltpu
from jax.experimental.pallas import tpu_sc as plsc

B0, B1 = 64, 32768
M = B0 * B1
JDIM, IDIM = 5, 4
E = JDIM * IDIM
K = 3
NC, NS, L = 2, 16, 16
NW = NC * NS
TG = B1 // 128
TGB = 8
CTOK = TGB * 128
NCH = B0 * (TG // TGB)
CPW = NCH // NW
JSTRIDE = TG * 512
GROUPS = CTOK // L


def _round_bf16(v):
    u = plsc.bitcast(v, jnp.int32)
    u = (u + 0x8000) & jnp.int32(-65536)
    return plsc.bitcast(u, jnp.float32)


def _sc_body(xf, wf, wbf, bf, vals, idxs, w_v, wb_v, b_v, in_v, vo_v, io_v):
    cid = lax.axis_index("c")
    sid = lax.axis_index("s")
    wid = sid * NC + cid
    pltpu.sync_copy(wf, w_v)
    pltpu.sync_copy(wbf, wb_v)
    pltpu.sync_copy(bf, b_v)
    lanes = lax.iota(jnp.int32, L)
    wsf = [w_v[pl.ds(k * L, L)] for k in range(E)]
    wsb = [wb_v[pl.ds(k * L, L)] for k in range(E)]
    bs = [b_v[pl.ds(o * L, L)] for o in range(JDIM)]

    def chunk_body(c, carry):
        ci = wid * CPW + c
        b = ci // (TG // TGB)
        tg0 = (ci % (TG // TGB)) * TGB
        src0 = (b * JDIM * TG + tg0) * 512
        for j in range(JDIM):
            pltpu.sync_copy(xf.at[pl.ds(src0 + j * JSTRIDE, TGB * 512)],
                            in_v.at[pl.ds(j * TGB * 512, TGB * 512)])

        def group_body(g, carry):
            tgl = g // 8
            g16 = g % 8
            base_t = tgl * 512 + g16 * L
            o3 = (g * L + lanes) * K
            s = []
            for j in range(JDIM):
                h = [_round_bf16(
                        in_v[pl.ds(j * TGB * 512 + base_t + i * 128, L)]
                        + wsf[j * IDIM + i])
                     for i in range(IDIM)]
                acc_sum = None
                for o in range(JDIM):
                    acc = bs[o]
                    for i in range(IDIM):
                        acc = acc + h[i] * wsb[o * IDIM + i]
                    acc = jnp.maximum(acc, 0.0)
                    acc_sum = acc if acc_sum is None else acc_sum + acc
                s.append(acc_sum)
            for k in range(K):
                bv = s[0]
                bi = jnp.zeros((L,), jnp.int32)
                for j in range(1, JDIM):
                    gt = s[j] > bv
                    bv = jnp.where(gt, s[j], bv)
                    bi = jnp.where(gt, j, bi)
                plsc.store_scatter(vo_v, [o3 + k], bv)
                plsc.store_scatter(io_v, [o3 + k], bi)
                if k < K - 1:
                    s = [jnp.where(bi == j, -1.0, s[j]) for j in range(JDIM)]
            return carry

        lax.fori_loop(0, GROUPS, group_body, 0)
        dst0 = (b * B1 + tg0 * 128) * K
        pltpu.sync_copy(vo_v, vals.at[pl.ds(dst0, CTOK * K)])
        pltpu.sync_copy(io_v, idxs.at[pl.ds(dst0, CTOK * K)])
        return carry

    lax.fori_loop(0, CPW, chunk_body, 0)


def kernel(x, W, b):
    xf = (x.transpose(0, 2, 3, 1)
           .reshape(B0, JDIM, IDIM, TG, 128)
           .transpose(0, 1, 3, 2, 4)
           .reshape(M * E))
    wi = lax.bitcast_convert_type(W, jnp.int32)
    wi = (wi + 0x7FFF + ((wi >> 16) & 1)) & jnp.int32(-65536)
    Wb = lax.bitcast_convert_type(wi, jnp.float32)
    wf = jnp.repeat(W.reshape(E), L)
    wbf = jnp.repeat(Wb.reshape(E), L)
    bf = jnp.repeat(b, L)
    mesh = plsc.VectorSubcoreMesh(core_axis_name="c", subcore_axis_name="s")
    vals, idxs = pl.kernel(
        _sc_body,
        out_type=(jax.ShapeDtypeStruct((M * K,), jnp.float32),
                  jax.ShapeDtypeStruct((M * K,), jnp.int32)),
        mesh=mesh,
        compiler_params=pltpu.CompilerParams(needs_layout_passes=False),
        scratch_types=[
            pltpu.VMEM((E * L,), jnp.float32),
            pltpu.VMEM((E * L,), jnp.float32),
            pltpu.VMEM((JDIM * L,), jnp.float32),
            pltpu.VMEM((JDIM * TGB * 512,), jnp.float32),
            pltpu.VMEM((CTOK * K,), jnp.float32),
            pltpu.VMEM((CTOK * K,), jnp.int32),
        ],
    )(xf, wf, wbf, bf)
    return vals.reshape(B0, B1, K), idxs.reshape(B0, B1, K)


# speedup vs baseline: 38.5110x; 6.5987x over previous
import jax
import jax.numpy as jnp
from jax import lax
from jax.experimental import pallas as pl
from jax.experimental.pallas import tpu as pltpu
from jax.experimental.pallas import tpu_sc as plsc

B0, B1 = 64, 32768
M = B0 * B1
JDIM, IDIM = 5, 4
E = JDIM * IDIM
K = 3
NC, NS, L = 2, 16, 16
NW = NC * NS
TG = B1 // 128
BG = 8
NBG = B0 // BG
TGB = 4
NCH = NBG * (TG // TGB)
CPW = NCH // NW
CTOK = BG * TGB * 128
GROUPS = CTOK // L
INW = BG * JDIM * TGB * 512
OUTW = K * TGB * BG * 128


def _round_bf16(v):
    u = plsc.bitcast(v, jnp.int32)
    u = (u + 0x8000) & jnp.int32(-65536)
    return plsc.bitcast(u, jnp.float32)


def _sc_body(xf, wf, wbf, bf, vals, idxs,
             w_v, wb_v, b_v, in_v, vo_v, io_v, sem):
    cid = lax.axis_index("c")
    sid = lax.axis_index("s")
    wid = sid * NC + cid
    pltpu.sync_copy(wf, w_v)
    pltpu.sync_copy(wbf, wb_v)
    pltpu.sync_copy(bf, b_v)
    lanes = lax.iota(jnp.int32, L)
    wsf = [w_v[pl.ds(k * L, L)] for k in range(E)]
    wsb = [wb_v[pl.ds(k * L, L)] for k in range(E)]
    bs = [b_v[pl.ds(o * L, L)] for o in range(JDIM)]

    def chunk_body(c, carry):
        ci = wid * CPW + c
        bg = ci // (TG // TGB)
        tg0 = (ci % (TG // TGB)) * TGB
        copies = []
        for bl in range(BG):
            b = bg * BG + bl
            for j in range(JDIM):
                src = ((b * JDIM + j) * TG + tg0) * 512
                dst = (bl * JDIM + j) * (TGB * 512)
                copies.append(pltpu.async_copy(
                    xf.at[pl.ds(src, TGB * 512)],
                    in_v.at[pl.ds(dst, TGB * 512)], sem))
        for cp in copies:
            cp.wait()

        def group_body(g, carry):
            bl = g // 32
            r = g - bl * 32
            tgl = r // 8
            g16 = r - tgl * 8
            ibase = bl * (JDIM * TGB * 512) + tgl * 512 + g16 * L
            obase = tgl * (BG * 128) + bl * 128 + g16 * L
            s = []
            for j in range(JDIM):
                h = [_round_bf16(
                        in_v[pl.ds(ibase + j * (TGB * 512) + i * 128, L)]
                        + wsf[j * IDIM + i])
                     for i in range(IDIM)]
                acc_sum = None
                for o in range(JDIM):
                    acc = bs[o]
                    for i in range(IDIM):
                        acc = acc + h[i] * wsb[o * IDIM + i]
                    acc = jnp.maximum(acc, 0.0)
                    acc_sum = acc if acc_sum is None else acc_sum + acc
                s.append(acc_sum)
            for k in range(K):
                bv = s[0]
                bi = jnp.zeros((L,), jnp.int32)
                for j in range(1, JDIM):
                    gt = s[j] > bv
                    bv = jnp.where(gt, s[j], bv)
                    bi = jnp.where(gt, j, bi)
                vo_v[pl.ds(k * (TGB * BG * 128) + obase, L)] = bv
                io_v[pl.ds(k * (TGB * BG * 128) + obase, L)] = bi
                if k < K - 1:
                    s = [jnp.where(bi == j, -1.0, s[j]) for j in range(JDIM)]
            return carry

        lax.fori_loop(0, GROUPS, group_body, 0)
        ocopies = []
        for k in range(K):
            dst = k * M + bg * (TG * BG * 128) + tg0 * (BG * 128)
            src = k * (TGB * BG * 128)
            ocopies.append(pltpu.async_copy(
                vo_v.at[pl.ds(src, TGB * BG * 128)],
                vals.at[pl.ds(dst, TGB * BG * 128)], sem))
            ocopies.append(pltpu.async_copy(
                io_v.at[pl.ds(src, TGB * BG * 128)],
                idxs.at[pl.ds(dst, TGB * BG * 128)], sem))
        for cp in ocopies:
            cp.wait()
        return carry

    lax.fori_loop(0, CPW, chunk_body, 0)


def kernel(x, W, b):
    xf = (x.transpose(0, 2, 3, 1)
           .reshape(B0, JDIM, IDIM, TG, 128)
           .transpose(0, 1, 3, 2, 4)
           .reshape(M * E))
    wi = lax.bitcast_convert_type(W, jnp.int32)
    wi = (wi + 0x7FFF + ((wi >> 16) & 1)) & jnp.int32(-65536)
    Wb = lax.bitcast_convert_type(wi, jnp.float32)
    wf = jnp.repeat(W.reshape(E), L)
    wbf = jnp.repeat(Wb.reshape(E), L)
    bf = jnp.repeat(b, L)
    mesh = plsc.VectorSubcoreMesh(core_axis_name="c", subcore_axis_name="s")
    vals, idxs = pl.kernel(
        _sc_body,
        out_type=(jax.ShapeDtypeStruct((M * K,), jnp.float32),
                  jax.ShapeDtypeStruct((M * K,), jnp.int32)),
        mesh=mesh,
        compiler_params=pltpu.CompilerParams(needs_layout_passes=False),
        scratch_types=[
            pltpu.VMEM((E * L,), jnp.float32),
            pltpu.VMEM((E * L,), jnp.float32),
            pltpu.VMEM((JDIM * L,), jnp.float32),
            pltpu.VMEM((INW,), jnp.float32),
            pltpu.VMEM((OUTW,), jnp.float32),
            pltpu.VMEM((OUTW,), jnp.int32),
            pltpu.SemaphoreType.DMA,
        ],
    )(xf, wf, wbf, bf)
    vals = (vals.reshape(K, NBG, TG, BG, 128)
                .transpose(1, 3, 2, 4, 0).reshape(B0, B1, K))
    idxs = (idxs.reshape(K, NBG, TG, BG, 128)
                .transpose(1, 3, 2, 4, 0).reshape(B0, B1, K))
    return vals, idxs
